# baseline (speedup 1.0000x reference)
import functools

import jax
import jax.numpy as jnp
from jax.experimental import pallas as pl
from jax.experimental.pallas import tpu as pltpu

N = 10000
D = 128
RB_A = 400
RB_B = 1000
F8 = jnp.float8_e4m3fn


def _deg_xs_kernel(adj_ref, x_ref, deg_ref, xs_hi_ref, xs_lo_ref, adj8_ref):
    a = adj_ref[...]
    deg = jnp.sum(a, axis=1, keepdims=True) + 1.0
    deg_ref[...] = deg
    xs = x_ref[...] * jax.lax.rsqrt(deg)
    xs_hi = xs.astype(F8)
    xs_hi_ref[...] = xs_hi
    xs_lo_ref[...] = (xs - xs_hi.astype(jnp.float32)).astype(F8)
    adj8_ref[...] = a.astype(F8)


def _spmm_kernel(params_ref, adj_ref, xs_hi_ref, xs_lo_ref, deg_ref, x_ref,
                 h0_ref, w_ref, out_ref):
    a = adj_ref[...]
    acc = (jnp.dot(a, xs_hi_ref[...], preferred_element_type=jnp.float32)
           + jnp.dot(a, xs_lo_ref[...], preferred_element_type=jnp.float32))
    theta = params_ref[0]
    alpha = params_ref[1]
    dinv_i = jax.lax.rsqrt(deg_ref[...])
    hi = dinv_i * acc + (dinv_i * dinv_i) * x_ref[...]
    support = (1.0 - alpha) * hi + alpha * h0_ref[...]
    out_ref[...] = (theta * jnp.dot(support, w_ref[...],
                                    preferred_element_type=jnp.float32)
                    + (1.0 - theta) * support)


@functools.partial(jax.jit, static_argnames=())
def _gcnii(x, adj, h0, w, theta, alpha):
    deg, xs_hi, xs_lo, adj8 = pl.pallas_call(
        _deg_xs_kernel,
        grid=(N // RB_A,),
        in_specs=[
            pl.BlockSpec((RB_A, N), lambda i: (i, 0)),
            pl.BlockSpec((RB_A, D), lambda i: (i, 0)),
        ],
        out_specs=[
            pl.BlockSpec((RB_A, 1), lambda i: (i, 0)),
            pl.BlockSpec((RB_A, D), lambda i: (i, 0)),
            pl.BlockSpec((RB_A, D), lambda i: (i, 0)),
            pl.BlockSpec((RB_A, N), lambda i: (i, 0)),
        ],
        out_shape=[
            jax.ShapeDtypeStruct((N, 1), jnp.float32),
            jax.ShapeDtypeStruct((N, D), F8),
            jax.ShapeDtypeStruct((N, D), F8),
            jax.ShapeDtypeStruct((N, N), F8),
        ],
        compiler_params=pltpu.CompilerParams(
            dimension_semantics=("parallel",),
        ),
    )(adj, x)

    params = jnp.stack([theta, alpha]).astype(jnp.float32)
    out = pl.pallas_call(
        _spmm_kernel,
        grid=(N // RB_B,),
        in_specs=[
            pl.BlockSpec(memory_space=pltpu.SMEM),
            pl.BlockSpec((RB_B, N), lambda i: (i, 0)),
            pl.BlockSpec((N, D), lambda i: (0, 0)),
            pl.BlockSpec((N, D), lambda i: (0, 0)),
            pl.BlockSpec((RB_B, 1), lambda i: (i, 0)),
            pl.BlockSpec((RB_B, D), lambda i: (i, 0)),
            pl.BlockSpec((RB_B, D), lambda i: (i, 0)),
            pl.BlockSpec((D, D), lambda i: (0, 0)),
        ],
        out_specs=pl.BlockSpec((RB_B, D), lambda i: (i, 0)),
        out_shape=jax.ShapeDtypeStruct((N, D), jnp.float32),
        compiler_params=pltpu.CompilerParams(
            dimension_semantics=("parallel",),
        ),
    )(params, adj8, xs_hi, xs_lo, deg, x, h0, w)
    return out


def kernel(input, adj, h0, W, lamda, alpha, l):
    theta = jnp.log(jnp.asarray(lamda, dtype=jnp.float32)
                    / jnp.asarray(l, dtype=jnp.float32) + 1.0)
    alpha = jnp.asarray(alpha, dtype=jnp.float32)
    return _gcnii(input, adj, h0, W, theta, alpha)


# speedup vs baseline: 1.0863x; 1.0863x over previous
import functools

import jax
import jax.numpy as jnp
from jax.experimental import pallas as pl
from jax.experimental.pallas import tpu as pltpu

N = 10000
D = 128
RB_A = 400
RB_B = 1000
F8 = jnp.float8_e4m3fn


def _deg_xs_kernel(adj_ref, x_ref, deg_ref, xs2_ref, adj8_ref):
    a = adj_ref[...]
    deg = jnp.sum(a, axis=1, keepdims=True) + 1.0
    deg_ref[...] = deg
    xs = x_ref[...] * jax.lax.rsqrt(deg)
    xs_hi = xs.astype(F8)
    xs_lo = (xs - xs_hi.astype(jnp.float32)).astype(F8)
    xs2_ref[...] = jnp.concatenate([xs_hi, xs_lo], axis=1)
    adj8_ref[...] = a.astype(F8)


def _spmm_kernel(params_ref, adj_ref, xs2_ref, deg_ref, x_ref,
                 h0_ref, w_ref, out_ref):
    a = adj_ref[...]
    prod = jnp.dot(a, xs2_ref[...], preferred_element_type=jnp.float32)
    acc = prod[:, :D] + prod[:, D:]
    theta = params_ref[0]
    alpha = params_ref[1]
    dinv_i = jax.lax.rsqrt(deg_ref[...])
    hi = dinv_i * acc + (dinv_i * dinv_i) * x_ref[...]
    support = (1.0 - alpha) * hi + alpha * h0_ref[...]
    out_ref[...] = (theta * jnp.dot(support, w_ref[...],
                                    preferred_element_type=jnp.float32)
                    + (1.0 - theta) * support)


@functools.partial(jax.jit, static_argnames=())
def _gcnii(x, adj, h0, w, theta, alpha):
    deg, xs2, adj8 = pl.pallas_call(
        _deg_xs_kernel,
        grid=(N // RB_A,),
        in_specs=[
            pl.BlockSpec((RB_A, N), lambda i: (i, 0)),
            pl.BlockSpec((RB_A, D), lambda i: (i, 0)),
        ],
        out_specs=[
            pl.BlockSpec((RB_A, 1), lambda i: (i, 0)),
            pl.BlockSpec((RB_A, 2 * D), lambda i: (i, 0)),
            pl.BlockSpec((RB_A, N), lambda i: (i, 0)),
        ],
        out_shape=[
            jax.ShapeDtypeStruct((N, 1), jnp.float32),
            jax.ShapeDtypeStruct((N, 2 * D), F8),
            jax.ShapeDtypeStruct((N, N), F8),
        ],
        compiler_params=pltpu.CompilerParams(
            dimension_semantics=("parallel",),
        ),
    )(adj, x)

    params = jnp.stack([theta, alpha]).astype(jnp.float32)
    out = pl.pallas_call(
        _spmm_kernel,
        grid=(N // RB_B,),
        in_specs=[
            pl.BlockSpec(memory_space=pltpu.SMEM),
            pl.BlockSpec((RB_B, N), lambda i: (i, 0)),
            pl.BlockSpec((N, 2 * D), lambda i: (0, 0)),
            pl.BlockSpec((RB_B, 1), lambda i: (i, 0)),
            pl.BlockSpec((RB_B, D), lambda i: (i, 0)),
            pl.BlockSpec((RB_B, D), lambda i: (i, 0)),
            pl.BlockSpec((D, D), lambda i: (0, 0)),
        ],
        out_specs=pl.BlockSpec((RB_B, D), lambda i: (i, 0)),
        out_shape=jax.ShapeDtypeStruct((N, D), jnp.float32),
        compiler_params=pltpu.CompilerParams(
            dimension_semantics=("parallel",),
        ),
    )(params, adj8, xs2, deg, x, h0, w)
    return out


def kernel(input, adj, h0, W, lamda, alpha, l):
    theta = jnp.log(jnp.asarray(lamda, dtype=jnp.float32)
                    / jnp.asarray(l, dtype=jnp.float32) + 1.0)
    alpha = jnp.asarray(alpha, dtype=jnp.float32)
    return _gcnii(input, adj, h0, W, theta, alpha)
